# baseline (speedup 1.0000x reference)
import jax
import jax.numpy as jnp
from jax import lax
from jax.experimental import pallas as pl
from jax.experimental.pallas import tpu as pltpu
from jax.experimental.pallas import tpu_sc as plsc

B, S, D, CMAX = 16, 4096, 256, 2048
KEEP = 50
L = 16
HALF = CMAX // 2
CH = 64
NCH = HALF // CH
NBUF = 6
DEPTH = 4
IDXN = DEPTH + 1
UNROLL = 4


def _body(table, tok, out, tok_v, pos_v, idx_v, gbuf, tsem, gsems, ssems):
    cid = lax.axis_index("c")
    sid = lax.axis_index("s")
    row = sid
    half = (cid + sid) % 2
    base = half * HALF

    tok_cp = pltpu.async_copy(tok.at[row], tok_v, tsem)

    zeros_i = jnp.zeros((L,), jnp.int32)
    zeros_f = jnp.zeros((L,), jnp.float32)

    def zfill(r, _):
        pos_v[pl.ds(r * L, L)] = zeros_i
        return 0
    lax.fori_loop(0, (CMAX + L) // L, zfill, 0)

    tok_cp.wait()

    iota = lax.iota(jnp.int32, L)
    rowbase = row * S

    def compact(v, cnt):
        for u in range(UNROLL):
            off = v * (L * UNROLL) + u * L
            t = tok_v[pl.ds(off, L)]
            m = t < KEEP
            mi = jnp.where(m, jnp.int32(1), jnp.int32(0))
            gidx = iota + (off + rowbase)
            dest = jnp.maximum(plsc.cumsum(mi) + cnt - 1, 0)
            plsc.store_scatter(pos_v, [dest], gidx, mask=m)
            cnt = cnt + plsc.all_reduce_population_count(m)
        return cnt

    cnt = lax.fori_loop(0, S // (L * UNROLL), compact, zeros_i)
    count = jnp.max(cnt)

    k = jnp.clip(jnp.minimum(count, CMAX) - base, 0, HALF)
    outbase = row * CMAX + base

    def gather(j):
        ib = idx_v.at[j % IDXN]
        for u in range(CH // L):
            ib[pl.ds(u * L, L)] = pos_v[pl.ds(base + j * CH + u * L, L)]
        bb = j % NBUF
        pltpu.async_copy(table.at[ib], gbuf.at[bb], gsems[bb])

    for j in range(DEPTH):
        gather(j)

    for j in range(NCH):
        b = j % NBUF
        gb = gbuf.at[b]
        dst = out.at[pl.ds(outbase + j * CH, CH)]
        pltpu.make_async_copy(table.at[idx_v.at[j % IDXN]], gb,
                              gsems[b]).wait()

        kj = jnp.clip(k - j * CH, 0, CH)

        def ztail(r, _):
            for u in range(D // L):
                gb[r, pl.ds(u * L, L)] = zeros_f
            return 0
        lax.fori_loop(kj, CH, ztail, 0)

        pltpu.async_copy(gb, dst, ssems[b])

        jj = j + DEPTH
        if jj < NCH:
            if jj >= NBUF:
                prev = out.at[pl.ds(outbase + (jj - NBUF) * CH, CH)]
                pltpu.make_async_copy(gbuf.at[jj % NBUF], prev,
                                      ssems[jj % NBUF]).wait()
            gather(jj)

    for j in range(max(NCH - NBUF, 0), NCH):
        b = j % NBUF
        dst = out.at[pl.ds(outbase + j * CH, CH)]
        pltpu.make_async_copy(gbuf.at[b], dst, ssems[b]).wait()


def kernel(input_embeddings, token_ids):
    table = input_embeddings.reshape(B * S, D)
    tok = token_ids.astype(jnp.int32)
    mesh = plsc.VectorSubcoreMesh(core_axis_name="c", subcore_axis_name="s")
    run = pl.kernel(
        _body,
        mesh=mesh,
        compiler_params=pltpu.CompilerParams(needs_layout_passes=False),
        out_type=jax.ShapeDtypeStruct((B * CMAX, D), jnp.float32),
        scratch_types=[
            pltpu.VMEM((S,), jnp.int32),
            pltpu.VMEM((S + L,), jnp.int32),
            pltpu.VMEM((IDXN, CH), jnp.int32),
            pltpu.VMEM((NBUF, CH, D), jnp.float32),
            pltpu.SemaphoreType.DMA,
            [pltpu.SemaphoreType.DMA] * NBUF,
            [pltpu.SemaphoreType.DMA] * NBUF,
        ],
    )
    out = run(table, tok)
    return out.reshape(B, CMAX, D)
